# baseline (speedup 1.0000x reference)
import jax
import jax.numpy as jnp
from jax.experimental import pallas as pl
from jax.experimental.pallas import tpu as pltpu

TOP_K = 2
NUM_EXPERTS = 64
D_MODEL = 768
N_TOKENS = 32768

BLOCK = 4096
SUB = 512


def _fused_body(tokens_ref, wg_ref, logits_ref, sel_ref, ew_ref, load_ref):
    i = pl.program_id(0)
    x = tokens_ref[...]
    w = wg_ref[...]
    logits = jax.lax.dot_general(
        w, x, (((0,), (1,)), ((), ())),
        preferred_element_type=jnp.float32)
    logits_ref[...] = logits

    eidx = jax.lax.broadcasted_iota(jnp.int32, (NUM_EXPERTS, SUB), 0)
    acc = jnp.zeros((NUM_EXPERTS, 1), jnp.float32)

    for j in range(BLOCK // SUB):
        lg = logits[:, j * SUB:(j + 1) * SUB]
        m1 = jnp.max(lg, axis=0, keepdims=True)
        i1 = jnp.min(jnp.where(lg == m1, eidx, NUM_EXPERTS), axis=0, keepdims=True)
        first = eidx == i1
        masked = jnp.where(first, -jnp.inf, lg)
        m2 = jnp.max(masked, axis=0, keepdims=True)
        i2 = jnp.min(jnp.where(masked == m2, eidx, NUM_EXPERTS), axis=0, keepdims=True)

        e = jnp.exp(m2 - m1)
        s = 1.0 / (1.0 + e)
        w1 = s
        w2 = e * s

        sel_ref[:, j * SUB:(j + 1) * SUB] = jnp.concatenate([i1, i2], axis=0)
        ew = jnp.where(first, w1, 0.0) + jnp.where(eidx == i2, w2, 0.0)
        ew_ref[:, j * SUB:(j + 1) * SUB] = ew
        acc = acc + jnp.sum((ew > 0.0).astype(jnp.float32), axis=1, keepdims=True)

    @pl.when(i == 0)
    def _():
        load_ref[...] = jnp.zeros_like(load_ref)

    load_ref[...] += acc


@jax.jit
def kernel(tokens, w_gate, w_noise):
    del w_noise
    grid = (N_TOKENS // BLOCK,)
    logits_t, sel_t, ew_t, load = pl.pallas_call(
        _fused_body,
        grid=grid,
        in_specs=[
            pl.BlockSpec((BLOCK, D_MODEL), lambda i: (i, 0)),
            pl.BlockSpec((D_MODEL, NUM_EXPERTS), lambda i: (0, 0)),
        ],
        out_specs=[
            pl.BlockSpec((NUM_EXPERTS, BLOCK), lambda i: (0, i)),
            pl.BlockSpec((TOP_K, BLOCK), lambda i: (0, i)),
            pl.BlockSpec((NUM_EXPERTS, BLOCK), lambda i: (0, i)),
            pl.BlockSpec((NUM_EXPERTS, 1), lambda i: (0, 0)),
        ],
        out_shape=[
            jax.ShapeDtypeStruct((NUM_EXPERTS, N_TOKENS), jnp.float32),
            jax.ShapeDtypeStruct((TOP_K, N_TOKENS), jnp.int32),
            jax.ShapeDtypeStruct((NUM_EXPERTS, N_TOKENS), jnp.float32),
            jax.ShapeDtypeStruct((NUM_EXPERTS, 1), jnp.float32),
        ],
    )(tokens, w_gate)
    return logits_t.T, sel_t.T, ew_t.T, load.reshape(NUM_EXPERTS)


# speedup vs baseline: 1.0069x; 1.0069x over previous
import jax
import jax.numpy as jnp
from jax.experimental import pallas as pl
from jax.experimental.pallas import tpu as pltpu

TOP_K = 2
NUM_EXPERTS = 64
D_MODEL = 768
N_TOKENS = 32768

BLOCK = 4096


def _fused_body(tokens_a_ref, tokens_b_ref, wg_ref, logits_ref, sel_ref, ew_ref, load_ref):
    i = pl.program_id(0)
    w = wg_ref[...]
    la = jax.lax.dot_general(
        w, tokens_a_ref[...], (((0,), (1,)), ((), ())),
        preferred_element_type=jnp.float32)
    lb = jax.lax.dot_general(
        w, tokens_b_ref[...], (((0,), (1,)), ((), ())),
        preferred_element_type=jnp.float32)
    logits = jnp.concatenate([la, lb], axis=1)
    logits_ref[...] = logits

    eidx = jax.lax.broadcasted_iota(jnp.int32, (NUM_EXPERTS, BLOCK), 0)
    m1 = jnp.max(logits, axis=0, keepdims=True)
    i1 = jnp.min(jnp.where(logits == m1, eidx, NUM_EXPERTS), axis=0, keepdims=True)
    first = eidx == i1
    masked = jnp.where(first, -jnp.inf, logits)
    m2 = jnp.max(masked, axis=0, keepdims=True)
    i2 = jnp.min(jnp.where(masked == m2, eidx, NUM_EXPERTS), axis=0, keepdims=True)

    e = jnp.exp(m2 - m1)
    s = 1.0 / (1.0 + e)
    w1 = s
    w2 = e * s

    sel_ref[...] = jnp.concatenate([i1, i2], axis=0)
    ew = jnp.where(first, w1, 0.0) + jnp.where(eidx == i2, w2, 0.0)
    ew_ref[...] = ew

    partial = jnp.sum((ew > 0.0).astype(jnp.float32), axis=1, keepdims=True)

    @pl.when(i == 0)
    def _():
        load_ref[...] = jnp.zeros_like(load_ref)

    load_ref[...] += partial


@jax.jit
def kernel(tokens, w_gate, w_noise):
    del w_noise
    grid = (N_TOKENS // BLOCK,)
    logits_t, sel_t, ew_t, load = pl.pallas_call(
        _fused_body,
        grid=grid,
        in_specs=[
            pl.BlockSpec((BLOCK // 2, D_MODEL), lambda i: (2 * i, 0)),
            pl.BlockSpec((BLOCK // 2, D_MODEL), lambda i: (2 * i + 1, 0)),
            pl.BlockSpec((D_MODEL, NUM_EXPERTS), lambda i: (0, 0)),
        ],
        out_specs=[
            pl.BlockSpec((NUM_EXPERTS, BLOCK), lambda i: (0, i)),
            pl.BlockSpec((TOP_K, BLOCK), lambda i: (0, i)),
            pl.BlockSpec((NUM_EXPERTS, BLOCK), lambda i: (0, i)),
            pl.BlockSpec((NUM_EXPERTS, 1), lambda i: (0, 0)),
        ],
        out_shape=[
            jax.ShapeDtypeStruct((NUM_EXPERTS, N_TOKENS), jnp.float32),
            jax.ShapeDtypeStruct((TOP_K, N_TOKENS), jnp.int32),
            jax.ShapeDtypeStruct((NUM_EXPERTS, N_TOKENS), jnp.float32),
            jax.ShapeDtypeStruct((NUM_EXPERTS, 1), jnp.float32),
        ],
    )(tokens, tokens, w_gate)
    return logits_t.T, sel_t.T, ew_t.T, load.reshape(NUM_EXPERTS)
